# Initial kernel scaffold; baseline (speedup 1.0000x reference)
import jax
import jax.numpy as jnp
from jax.experimental import pallas as pl


def kernel(x, edge_index, W1, a_src1, a_dst1, b1, W2, a_src2, a_dst2, b2):
    raise NotImplementedError("write your pallas kernel here")


# speedup vs baseline: 17.3022x; 17.3022x over previous
import functools

import jax
import jax.numpy as jnp
from jax import lax
from jax.experimental import pallas as pl
from jax.experimental.pallas import tpu as pltpu
from jax.experimental.pallas import tpu_sc as plsc

N = 10000
D = 128
NC = 40
E = 320000

T1 = 128
A1 = 80
T2 = 128
A2 = 64

NP = 10240
ROWS = NP // 16
NPAD_ROWS = NP - N

NTILES = 32
CHUNK = 16
CHUNKS_PER_TILE = 648
PER_TILE = CHUNK * CHUNKS_PER_TILE
EPAD = NTILES * PER_TILE

_mesh = plsc.VectorSubcoreMesh(core_axis_name="c", subcore_axis_name="s")


def _tc_prep1(x_ref, w1_ref, as_ref, adr_ref, tab_ref):
    h = jnp.dot(x_ref[...], w1_ref[...], preferred_element_type=jnp.float32)
    a_s = jnp.dot(h, as_ref[...], preferred_element_type=jnp.float32)
    a_dr = jnp.dot(h, adr_ref[...], preferred_element_type=jnp.float32)
    tab_ref[:, 0:64] = h
    tab_ref[:, 64:72] = a_s
    tab_ref[:, 72:80] = a_dr
    tab_ref[:, 80:128] = jnp.zeros((NP, 48), jnp.float32)


def _tc_mid(acc_ref, b1_ref, w2_ref, asv_ref, adv_ref, e8_ref, tab_ref):
    acc = acc_ref[0] + acc_ref[1]
    num = acc[:, 0:64]
    den8 = acc[:, 64:72]
    denx = jnp.dot(den8, e8_ref[...],
                   preferred_element_type=jnp.float32)
    gat = num / (denx + 1e-16) + b1_ref[...]
    act = jnp.where(gat > 0.0, gat, jnp.exp(gat) - 1.0)
    h2 = jnp.dot(act, w2_ref[...], preferred_element_type=jnp.float32)
    as2 = jnp.dot(h2, asv_ref[...], preferred_element_type=jnp.float32)
    ad2 = jnp.dot(h2, adv_ref[...], preferred_element_type=jnp.float32)
    z = jnp.zeros((NP, 8), jnp.float32)
    tab_ref[:, 0:40] = h2
    tab_ref[:, 40:48] = z
    tab_ref[:, 48:49] = as2
    tab_ref[:, 49:50] = z[:, 0:1]
    tab_ref[:, 50:56] = z[:, 0:6]
    tab_ref[:, 56:63] = z[:, 0:7]
    tab_ref[:, 63:64] = ad2
    tab_ref[:, 64:128] = jnp.zeros((NP, 64), jnp.float32)


def _tc_final(acc_ref, b2_ref, o_ref):
    acc = acc_ref[0, 0:N] + acc_ref[1, 0:N]
    num = acc[:, 0:40]
    den = acc[:, 48:49]
    o = num / (den + 1e-16) + b2_ref[...]
    m = jnp.max(o, axis=1, keepdims=True)
    o = o - m
    o_ref[...] = o - jnp.log(jnp.sum(jnp.exp(o), axis=1, keepdims=True))


def _sc_layer(src_hbm, dst_hbm, tab_hbm, z_hbm, out_acc,
              idx_s, idx_d, buf_s, buf_d, msgb,
              acc_sh, *, aw, eoff, nmsg, nheads):
    c = lax.axis_index("c")
    s = lax.axis_index("s")
    wid = s * 2 + c
    rbase = s * ROWS

    pltpu.sync_copy(z_hbm, msgb)
    for t in range(ROWS // CHUNK):
        pltpu.sync_copy(msgb, acc_sh.at[pl.ds(rbase + t * CHUNK, CHUNK)])
    plsc.subcore_barrier()

    lane = lax.iota(jnp.int32, 16)
    lo = lane < 8

    def chunk_body(i, carry):
        ebase = wid * PER_TILE + i * CHUNK
        pltpu.sync_copy(src_hbm.at[pl.ds(ebase, CHUNK)], idx_s)
        pltpu.sync_copy(dst_hbm.at[pl.ds(ebase, CHUNK)], idx_d)
        iv_s = idx_s[...]
        iv_d = idx_d[...]
        pltpu.sync_copy(tab_hbm.at[iv_s], buf_s)
        pltpu.sync_copy(tab_hbm.at[iv_d], buf_d)

        def ebody(k, carry2):
            dvr = lax.rev(buf_d[k, pl.ds(eoff, 16)], (0,))
            ev = buf_s[k, pl.ds(eoff, 16)] + dvr
            ev = jnp.where(ev >= 0.0, ev, 0.2 * ev)
            wv = jnp.exp(ev)
            if nheads == 8:
                msgb[k, pl.ds(nmsg, 16)] = wv
                for j in range(4):
                    m = jnp.where(lo, wv[2 * j], wv[2 * j + 1])
                    msgb[k, pl.ds(16 * j, 16)] = buf_s[k, pl.ds(16 * j, 16)] * m
            else:
                msgb[k, pl.ds(nmsg, 16)] = jnp.where(lane == 0, wv, 0.0)
                w0 = wv[0]
                for j in range(3):
                    msgb[k, pl.ds(16 * j, 16)] = buf_s[k, pl.ds(16 * j, 16)] * w0
            return carry2

        lax.fori_loop(0, CHUNK, ebody, 0)
        pltpu.sync_copy(msgb, acc_sh.at[iv_d], add=True)
        return carry

    lax.fori_loop(0, CHUNKS_PER_TILE, chunk_body, 0)
    plsc.subcore_barrier()

    for t in range(ROWS // CHUNK):
        pltpu.sync_copy(acc_sh.at[pl.ds(rbase + t * CHUNK, CHUNK)], msgb)
        pltpu.sync_copy(msgb, out_acc.at[c, pl.ds(rbase + t * CHUNK, CHUNK)])


def _make_sc_layer(tw, aw, eoff, nmsg, nheads):
    body = functools.partial(
        _sc_layer, aw=aw, eoff=eoff, nmsg=nmsg, nheads=nheads)
    return functools.partial(
        pl.kernel,
        out_type=[jax.ShapeDtypeStruct((2, NP, aw), jnp.float32)],
        mesh=_mesh,
        scratch_types=[
            pltpu.VMEM((CHUNK,), jnp.int32),
            pltpu.VMEM((CHUNK,), jnp.int32),
            pltpu.VMEM((CHUNK, tw), jnp.float32),
            pltpu.VMEM((CHUNK, tw), jnp.float32),
            pltpu.VMEM((CHUNK, aw), jnp.float32),
            pltpu.VMEM_SHARED((NP, aw), jnp.float32),
        ],
    )(body)


_layer1_edges = _make_sc_layer(tw=T1, aw=A1, eoff=64, nmsg=64, nheads=8)
_layer2_edges = _make_sc_layer(tw=T2, aw=A2, eoff=48, nmsg=48, nheads=1)


@jax.jit
def kernel(x, edge_index, W1, a_src1, a_dst1, b1, W2, a_src2, a_dst2, b2):
    f32 = jnp.float32
    x_p = jnp.pad(x, ((0, NP - N), (0, 0)))
    loops = jnp.arange(N, dtype=jnp.int32)
    npad = EPAD - (E + N)
    pad_idx = N + (jnp.arange(npad, dtype=jnp.int32) % NPAD_ROWS)
    src = jnp.concatenate([edge_index[0].astype(jnp.int32), loops, pad_idx])
    dst = jnp.concatenate([edge_index[1].astype(jnp.int32), loops, pad_idx])

    eye8 = jnp.eye(8, dtype=f32)
    As1 = jnp.einsum("hc,hg->hcg", a_src1, eye8).reshape(64, 8)
    Ad1 = jnp.einsum("hc,hg->hcg", a_dst1, eye8).reshape(64, 8)[:, ::-1]
    E8 = jnp.repeat(eye8, 8, axis=1)

    z1 = jnp.zeros((CHUNK, A1), f32)
    z2 = jnp.zeros((CHUNK, A2), f32)

    tab1 = pl.pallas_call(
        _tc_prep1,
        out_shape=jax.ShapeDtypeStruct((NP, T1), f32),
    )(x_p, W1, As1, Ad1)

    acc1, = _layer1_edges(src, dst, tab1, z1)

    tab2 = pl.pallas_call(
        _tc_mid,
        out_shape=jax.ShapeDtypeStruct((NP, T2), f32),
    )(acc1, b1, W2, a_src2.reshape(NC, 1), a_dst2.reshape(NC, 1), E8)

    acc2, = _layer2_edges(src, dst, tab2, z2)

    out = pl.pallas_call(
        _tc_final,
        out_shape=jax.ShapeDtypeStruct((N, NC), f32),
    )(acc2, b2)
    return out


# speedup vs baseline: 17.4283x; 1.0073x over previous
import functools

import jax
import jax.numpy as jnp
from jax import lax
from jax.experimental import pallas as pl
from jax.experimental.pallas import tpu as pltpu
from jax.experimental.pallas import tpu_sc as plsc

N = 10000
D = 128
NC = 40
E = 320000

T1 = 128
A1 = 80
T2 = 128
A2 = 64

NP = 10240
ROWS = NP // 16
NPAD_ROWS = NP - N

NTILES = 32
CHUNK = 16
CHUNKS_PER_TILE = 648
PER_TILE = CHUNK * CHUNKS_PER_TILE
EPAD = NTILES * PER_TILE

_mesh = plsc.VectorSubcoreMesh(core_axis_name="c", subcore_axis_name="s")


def _tc_prep1(x_ref, w1_ref, as_ref, adr_ref, tab_ref):
    h = jnp.dot(x_ref[...], w1_ref[...], preferred_element_type=jnp.float32)
    a_s = jnp.dot(h, as_ref[...], preferred_element_type=jnp.float32)
    a_dr = jnp.dot(h, adr_ref[...], preferred_element_type=jnp.float32)
    tab_ref[:, 0:64] = h
    tab_ref[:, 64:72] = a_s
    tab_ref[:, 72:80] = a_dr
    tab_ref[:, 80:128] = jnp.zeros((NP, 48), jnp.float32)


def _tc_mid(acc_ref, b1_ref, w2_ref, asv_ref, adv_ref, e8_ref, tab_ref):
    acc = acc_ref[0] + acc_ref[1]
    num = acc[:, 0:64]
    den8 = acc[:, 64:72]
    denx = jnp.dot(den8, e8_ref[...],
                   preferred_element_type=jnp.float32)
    gat = num / (denx + 1e-16) + b1_ref[...]
    act = jnp.where(gat > 0.0, gat, jnp.exp(gat) - 1.0)
    h2 = jnp.dot(act, w2_ref[...], preferred_element_type=jnp.float32)
    as2 = jnp.dot(h2, asv_ref[...], preferred_element_type=jnp.float32)
    ad2 = jnp.dot(h2, adv_ref[...], preferred_element_type=jnp.float32)
    z = jnp.zeros((NP, 8), jnp.float32)
    tab_ref[:, 0:40] = h2
    tab_ref[:, 40:48] = z
    tab_ref[:, 48:49] = as2
    tab_ref[:, 49:50] = z[:, 0:1]
    tab_ref[:, 50:56] = z[:, 0:6]
    tab_ref[:, 56:63] = z[:, 0:7]
    tab_ref[:, 63:64] = ad2
    tab_ref[:, 64:128] = jnp.zeros((NP, 64), jnp.float32)


def _tc_final(acc_ref, b2_ref, o_ref):
    acc = acc_ref[0, 0:N] + acc_ref[1, 0:N]
    num = acc[:, 0:40]
    den = acc[:, 48:49]
    o = num / (den + 1e-16) + b2_ref[...]
    m = jnp.max(o, axis=1, keepdims=True)
    o = o - m
    o_ref[...] = o - jnp.log(jnp.sum(jnp.exp(o), axis=1, keepdims=True))


def _sc_layer(src_hbm, dst_hbm, tab_hbm, z_hbm, out_acc,
              idx_s, idx_d, buf_s, buf_d, msgb,
              acc_sh, *, aw, eoff, nmsg, nheads):
    c = lax.axis_index("c")
    s = lax.axis_index("s")
    wid = s * 2 + c
    rbase = s * ROWS

    pltpu.sync_copy(z_hbm, msgb)
    for t in range(ROWS // CHUNK):
        pltpu.sync_copy(msgb, acc_sh.at[pl.ds(rbase + t * CHUNK, CHUNK)])
    plsc.subcore_barrier()

    lane = lax.iota(jnp.int32, 16)
    lo = lane < 8

    def chunk_body(i, carry):
        ebase = wid * PER_TILE + i * CHUNK
        pltpu.sync_copy(src_hbm.at[pl.ds(ebase, CHUNK)], idx_s)
        pltpu.sync_copy(dst_hbm.at[pl.ds(ebase, CHUNK)], idx_d)
        iv_s = idx_s[...]
        iv_d = idx_d[...]
        pltpu.sync_copy(tab_hbm.at[iv_s], buf_s)
        pltpu.sync_copy(tab_hbm.at[iv_d], buf_d)

        def ebody(k):
            dvr = lax.rev(buf_d[k, pl.ds(eoff, 16)], (0,))
            ev = buf_s[k, pl.ds(eoff, 16)] + dvr
            ev = jnp.where(ev >= 0.0, ev, 0.2 * ev)
            wv = jnp.exp(ev)
            if nheads == 8:
                msgb[k, pl.ds(nmsg, 16)] = wv
                for j in range(4):
                    m = jnp.where(lo, wv[2 * j], wv[2 * j + 1])
                    msgb[k, pl.ds(16 * j, 16)] = buf_s[k, pl.ds(16 * j, 16)] * m
            else:
                msgb[k, pl.ds(nmsg, 16)] = jnp.where(lane == 0, wv, 0.0)
                w0 = wv[0]
                for j in range(3):
                    msgb[k, pl.ds(16 * j, 16)] = buf_s[k, pl.ds(16 * j, 16)] * w0

        for k in range(CHUNK):
            ebody(k)
        pltpu.sync_copy(msgb, acc_sh.at[iv_d], add=True)
        return carry

    lax.fori_loop(0, CHUNKS_PER_TILE, chunk_body, 0)
    plsc.subcore_barrier()

    for t in range(ROWS // CHUNK):
        pltpu.sync_copy(acc_sh.at[pl.ds(rbase + t * CHUNK, CHUNK)], msgb)
        pltpu.sync_copy(msgb, out_acc.at[c, pl.ds(rbase + t * CHUNK, CHUNK)])


def _make_sc_layer(tw, aw, eoff, nmsg, nheads):
    body = functools.partial(
        _sc_layer, aw=aw, eoff=eoff, nmsg=nmsg, nheads=nheads)
    return functools.partial(
        pl.kernel,
        out_type=[jax.ShapeDtypeStruct((2, NP, aw), jnp.float32)],
        mesh=_mesh,
        scratch_types=[
            pltpu.VMEM((CHUNK,), jnp.int32),
            pltpu.VMEM((CHUNK,), jnp.int32),
            pltpu.VMEM((CHUNK, tw), jnp.float32),
            pltpu.VMEM((CHUNK, tw), jnp.float32),
            pltpu.VMEM((CHUNK, aw), jnp.float32),
            pltpu.VMEM_SHARED((NP, aw), jnp.float32),
        ],
    )(body)


_layer1_edges = _make_sc_layer(tw=T1, aw=A1, eoff=64, nmsg=64, nheads=8)
_layer2_edges = _make_sc_layer(tw=T2, aw=A2, eoff=48, nmsg=48, nheads=1)


@jax.jit
def kernel(x, edge_index, W1, a_src1, a_dst1, b1, W2, a_src2, a_dst2, b2):
    f32 = jnp.float32
    x_p = jnp.pad(x, ((0, NP - N), (0, 0)))
    loops = jnp.arange(N, dtype=jnp.int32)
    npad = EPAD - (E + N)
    pad_idx = N + (jnp.arange(npad, dtype=jnp.int32) % NPAD_ROWS)
    src = jnp.concatenate([edge_index[0].astype(jnp.int32), loops, pad_idx])
    dst = jnp.concatenate([edge_index[1].astype(jnp.int32), loops, pad_idx])

    eye8 = jnp.eye(8, dtype=f32)
    As1 = jnp.einsum("hc,hg->hcg", a_src1, eye8).reshape(64, 8)
    Ad1 = jnp.einsum("hc,hg->hcg", a_dst1, eye8).reshape(64, 8)[:, ::-1]
    E8 = jnp.repeat(eye8, 8, axis=1)

    z1 = jnp.zeros((CHUNK, A1), f32)
    z2 = jnp.zeros((CHUNK, A2), f32)

    tab1 = pl.pallas_call(
        _tc_prep1,
        out_shape=jax.ShapeDtypeStruct((NP, T1), f32),
    )(x_p, W1, As1, Ad1)

    acc1, = _layer1_edges(src, dst, tab1, z1)

    tab2 = pl.pallas_call(
        _tc_mid,
        out_shape=jax.ShapeDtypeStruct((NP, T2), f32),
    )(acc1, b1, W2, a_src2.reshape(NC, 1), a_dst2.reshape(NC, 1), E8)

    acc2, = _layer2_edges(src, dst, tab2, z2)

    out = pl.pallas_call(
        _tc_final,
        out_shape=jax.ShapeDtypeStruct((N, NC), f32),
    )(acc2, b2)
    return out


# speedup vs baseline: 57.3883x; 3.2928x over previous
import functools

import jax
import jax.numpy as jnp
from jax import lax
from jax.experimental import pallas as pl
from jax.experimental.pallas import tpu as pltpu
from jax.experimental.pallas import tpu_sc as plsc

N = 10000
D = 128
NC = 40
E = 320000

T1 = 128
A1 = 80
T2 = 128
A2 = 64

NP = 10240
ROWS = NP // 16
NPAD_ROWS = NP - N

NTILES = 32
CHUNK = 16
CHUNKS_PER_TILE = 648
PER_TILE = CHUNK * CHUNKS_PER_TILE
EPAD = NTILES * PER_TILE

_mesh = plsc.VectorSubcoreMesh(core_axis_name="c", subcore_axis_name="s")


def _tc_prep1(x_ref, w1_ref, as_ref, adr_ref, tab_ref):
    h = jnp.dot(x_ref[...], w1_ref[...], preferred_element_type=jnp.float32)
    a_s = jnp.dot(h, as_ref[...], preferred_element_type=jnp.float32)
    a_dr = jnp.dot(h, adr_ref[...], preferred_element_type=jnp.float32)
    tab_ref[:, 0:64] = h
    tab_ref[:, 64:72] = a_s
    tab_ref[:, 72:80] = a_dr
    tab_ref[:, 80:128] = jnp.zeros((NP, 48), jnp.float32)


def _tc_mid(acc_ref, b1_ref, w2_ref, asv_ref, adv_ref, e8_ref, tab_ref):
    acc = acc_ref[0] + acc_ref[1]
    num = acc[:, 0:64]
    den8 = acc[:, 64:72]
    denx = jnp.dot(den8, e8_ref[...],
                   preferred_element_type=jnp.float32)
    gat = num / (denx + 1e-16) + b1_ref[...]
    act = jnp.where(gat > 0.0, gat, jnp.exp(gat) - 1.0)
    h2 = jnp.dot(act, w2_ref[...], preferred_element_type=jnp.float32)
    as2 = jnp.dot(h2, asv_ref[...], preferred_element_type=jnp.float32)
    ad2 = jnp.dot(h2, adv_ref[...], preferred_element_type=jnp.float32)
    z = jnp.zeros((NP, 8), jnp.float32)
    tab_ref[:, 0:40] = h2
    tab_ref[:, 40:48] = z
    tab_ref[:, 48:49] = as2
    tab_ref[:, 49:50] = z[:, 0:1]
    tab_ref[:, 50:56] = z[:, 0:6]
    tab_ref[:, 56:63] = z[:, 0:7]
    tab_ref[:, 63:64] = ad2
    tab_ref[:, 64:128] = jnp.zeros((NP, 64), jnp.float32)


def _tc_final(acc_ref, b2_ref, o_ref):
    acc = acc_ref[0, 0:N] + acc_ref[1, 0:N]
    num = acc[:, 0:40]
    den = acc[:, 48:49]
    o = num / (den + 1e-16) + b2_ref[...]
    m = jnp.max(o, axis=1, keepdims=True)
    o = o - m
    o_ref[...] = o - jnp.log(jnp.sum(jnp.exp(o), axis=1, keepdims=True))


def _sc_layer(src_hbm, dst_hbm, tab_hbm, z_hbm, out_acc,
              idx_s, idx_d, buf_s, buf_d, buf_s2, buf_d2, msgb,
              acc_sh, sem1, sem2, sem3, sem4, *, aw, eoff, nmsg, nheads):
    c = lax.axis_index("c")
    s = lax.axis_index("s")
    wid = s * 2 + c
    rbase = s * ROWS

    pltpu.sync_copy(src_hbm.at[pl.ds(wid * PER_TILE, PER_TILE)], idx_s)
    pltpu.sync_copy(dst_hbm.at[pl.ds(wid * PER_TILE, PER_TILE)], idx_d)

    pltpu.sync_copy(z_hbm, msgb)
    for t in range(ROWS // CHUNK):
        pltpu.sync_copy(msgb, acc_sh.at[pl.ds(rbase + t * CHUNK, CHUNK)])
    plsc.subcore_barrier()

    lane = lax.iota(jnp.int32, 16)
    lo = lane < 8

    def ebody(k, bs, bd):
        dvr = lax.rev(bd[k, pl.ds(eoff, 16)], (0,))
        ev = bs[k, pl.ds(eoff, 16)] + dvr
        ev = jnp.where(ev >= 0.0, ev, 0.2 * ev)
        wv = jnp.exp(ev)
        if nheads == 8:
            msgb[k, pl.ds(nmsg, 16)] = wv
            for j in range(4):
                m = jnp.where(lo, wv[2 * j], wv[2 * j + 1])
                msgb[k, pl.ds(16 * j, 16)] = bs[k, pl.ds(16 * j, 16)] * m
        else:
            msgb[k, pl.ds(nmsg, 16)] = jnp.where(lane == 0, wv, 0.0)
            w0 = wv[0]
            for j in range(3):
                msgb[k, pl.ds(16 * j, 16)] = bs[k, pl.ds(16 * j, 16)] * w0

    def chunk_pair(t, carry):
        c0 = 2 * t
        c1 = 2 * t + 1
        iv_s0 = idx_s[pl.ds(16 * c0, 16)]
        iv_d0 = idx_d[pl.ds(16 * c0, 16)]
        iv_s1 = idx_s[pl.ds(16 * c1, 16)]
        iv_d1 = idx_d[pl.ds(16 * c1, 16)]
        cp0s = pltpu.async_copy(tab_hbm.at[iv_s0], buf_s, sem1)
        cp0d = pltpu.async_copy(tab_hbm.at[iv_d0], buf_d, sem2)
        cp1s = pltpu.async_copy(tab_hbm.at[iv_s1], buf_s2, sem3)
        cp1d = pltpu.async_copy(tab_hbm.at[iv_d1], buf_d2, sem4)
        cp0s.wait()
        cp0d.wait()
        for k in range(CHUNK):
            ebody(k, buf_s, buf_d)
        pltpu.sync_copy(msgb, acc_sh.at[iv_d0], add=True)
        cp1s.wait()
        cp1d.wait()
        for k in range(CHUNK):
            ebody(k, buf_s2, buf_d2)
        pltpu.sync_copy(msgb, acc_sh.at[iv_d1], add=True)
        return carry

    lax.fori_loop(0, CHUNKS_PER_TILE // 2, chunk_pair, 0)
    plsc.subcore_barrier()

    for t in range(ROWS // CHUNK):
        pltpu.sync_copy(acc_sh.at[pl.ds(rbase + t * CHUNK, CHUNK)], msgb)
        pltpu.sync_copy(msgb, out_acc.at[c, pl.ds(rbase + t * CHUNK, CHUNK)])


def _make_sc_layer(tw, aw, eoff, nmsg, nheads):
    body = functools.partial(
        _sc_layer, aw=aw, eoff=eoff, nmsg=nmsg, nheads=nheads)
    return functools.partial(
        pl.kernel,
        out_type=[jax.ShapeDtypeStruct((2, NP, aw), jnp.float32)],
        mesh=_mesh,
        scratch_types=[
            pltpu.VMEM((PER_TILE,), jnp.int32),
            pltpu.VMEM((PER_TILE,), jnp.int32),
            pltpu.VMEM((CHUNK, tw), jnp.float32),
            pltpu.VMEM((CHUNK, tw), jnp.float32),
            pltpu.VMEM((CHUNK, tw), jnp.float32),
            pltpu.VMEM((CHUNK, tw), jnp.float32),
            pltpu.VMEM((CHUNK, aw), jnp.float32),
            pltpu.VMEM_SHARED((NP, aw), jnp.float32),
            pltpu.SemaphoreType.DMA,
            pltpu.SemaphoreType.DMA,
            pltpu.SemaphoreType.DMA,
            pltpu.SemaphoreType.DMA,
        ],
    )(body)


_layer1_edges = _make_sc_layer(tw=T1, aw=A1, eoff=64, nmsg=64, nheads=8)
_layer2_edges = _make_sc_layer(tw=T2, aw=A2, eoff=48, nmsg=48, nheads=1)


@jax.jit
def kernel(x, edge_index, W1, a_src1, a_dst1, b1, W2, a_src2, a_dst2, b2):
    f32 = jnp.float32
    x_p = jnp.pad(x, ((0, NP - N), (0, 0)))
    loops = jnp.arange(N, dtype=jnp.int32)
    npad = EPAD - (E + N)
    pad_idx = N + (jnp.arange(npad, dtype=jnp.int32) % NPAD_ROWS)
    src = jnp.concatenate([edge_index[0].astype(jnp.int32), loops, pad_idx])
    dst = jnp.concatenate([edge_index[1].astype(jnp.int32), loops, pad_idx])

    eye8 = jnp.eye(8, dtype=f32)
    As1 = jnp.einsum("hc,hg->hcg", a_src1, eye8).reshape(64, 8)
    Ad1 = jnp.einsum("hc,hg->hcg", a_dst1, eye8).reshape(64, 8)[:, ::-1]
    E8 = jnp.repeat(eye8, 8, axis=1)

    z1 = jnp.zeros((CHUNK, A1), f32)
    z2 = jnp.zeros((CHUNK, A2), f32)

    tab1 = pl.pallas_call(
        _tc_prep1,
        out_shape=jax.ShapeDtypeStruct((NP, T1), f32),
    )(x_p, W1, As1, Ad1)

    acc1, = _layer1_edges(src, dst, tab1, z1)

    tab2 = pl.pallas_call(
        _tc_mid,
        out_shape=jax.ShapeDtypeStruct((NP, T2), f32),
    )(acc1, b1, W2, a_src2.reshape(NC, 1), a_dst2.reshape(NC, 1), E8)

    acc2, = _layer2_edges(src, dst, tab2, z2)

    out = pl.pallas_call(
        _tc_final,
        out_shape=jax.ShapeDtypeStruct((N, NC), f32),
    )(acc2, b2)
    return out


# speedup vs baseline: 76.5759x; 1.3343x over previous
import functools

import jax
import jax.numpy as jnp
from jax import lax
from jax.experimental import pallas as pl
from jax.experimental.pallas import tpu as pltpu
from jax.experimental.pallas import tpu_sc as plsc

N = 10000
D = 128
NC = 40
E = 320000

T1 = 128
A1 = 80
T2 = 128
A2 = 64

NP = 10240
ROWS = NP // 16
NPAD_ROWS = NP - N

NTILES = 32
CHUNK = 16
CHUNKS_PER_TILE = 648
PER_TILE = CHUNK * CHUNKS_PER_TILE
EPAD = NTILES * PER_TILE

_mesh = plsc.VectorSubcoreMesh(core_axis_name="c", subcore_axis_name="s")


def _tc_prep1(x_ref, w1_ref, as_ref, adr_ref, tab_ref):
    h = jnp.dot(x_ref[...], w1_ref[...], preferred_element_type=jnp.float32)
    a_s = jnp.dot(h, as_ref[...], preferred_element_type=jnp.float32)
    a_dr = jnp.dot(h, adr_ref[...], preferred_element_type=jnp.float32)
    tab_ref[:, 0:64] = h
    tab_ref[:, 64:72] = a_s
    tab_ref[:, 72:80] = a_dr
    tab_ref[:, 80:128] = jnp.zeros((NP, 48), jnp.float32)


def _tc_mid(acc_ref, b1_ref, w2_ref, asv_ref, adv_ref, e8_ref, tab_ref):
    acc = acc_ref[0] + acc_ref[1]
    num = acc[:, 0:64]
    den8 = acc[:, 64:72]
    denx = jnp.dot(den8, e8_ref[...],
                   preferred_element_type=jnp.float32)
    gat = num / (denx + 1e-16) + b1_ref[...]
    act = jnp.where(gat > 0.0, gat, jnp.exp(gat) - 1.0)
    h2 = jnp.dot(act, w2_ref[...], preferred_element_type=jnp.float32)
    as2 = jnp.dot(h2, asv_ref[...], preferred_element_type=jnp.float32)
    ad2 = jnp.dot(h2, adv_ref[...], preferred_element_type=jnp.float32)
    z = jnp.zeros((NP, 8), jnp.float32)
    tab_ref[:, 0:40] = h2
    tab_ref[:, 40:48] = z
    tab_ref[:, 48:49] = as2
    tab_ref[:, 49:50] = z[:, 0:1]
    tab_ref[:, 50:56] = z[:, 0:6]
    tab_ref[:, 56:63] = z[:, 0:7]
    tab_ref[:, 63:64] = ad2
    tab_ref[:, 64:128] = jnp.zeros((NP, 64), jnp.float32)


def _tc_final(acc_ref, b2_ref, o_ref):
    acc = acc_ref[0, 0:N] + acc_ref[1, 0:N]
    num = acc[:, 0:40]
    den = acc[:, 48:49]
    o = num / (den + 1e-16) + b2_ref[...]
    m = jnp.max(o, axis=1, keepdims=True)
    o = o - m
    o_ref[...] = o - jnp.log(jnp.sum(jnp.exp(o), axis=1, keepdims=True))


def _sc_layer(src_hbm, dst_hbm, tab_hbm, z_hbm, out_acc,
              idx_s, idx_d, bs0, bd0, bs1, bd1, bs2, bd2, bs3, bd3, msgb,
              acc_sh, ss0, sd0, ss1, sd1, ss2, sd2, ss3, sd3,
              *, aw, eoff, nmsg, nheads):
    c = lax.axis_index("c")
    s = lax.axis_index("s")
    wid = s * 2 + c
    rbase = s * ROWS

    pltpu.sync_copy(src_hbm.at[pl.ds(wid * PER_TILE, PER_TILE)], idx_s)
    pltpu.sync_copy(dst_hbm.at[pl.ds(wid * PER_TILE, PER_TILE)], idx_d)

    pltpu.sync_copy(z_hbm, msgb)
    for t in range(ROWS // CHUNK):
        pltpu.sync_copy(msgb, acc_sh.at[pl.ds(rbase + t * CHUNK, CHUNK)])
    plsc.subcore_barrier()

    lane = lax.iota(jnp.int32, 16)
    lo = lane < 8

    def ebody(k, bs, bd):
        dvr = lax.rev(bd[k, pl.ds(eoff, 16)], (0,))
        ev = bs[k, pl.ds(eoff, 16)] + dvr
        ev = jnp.where(ev >= 0.0, ev, 0.2 * ev)
        wv = jnp.exp(ev)
        if nheads == 8:
            msgb[k, pl.ds(nmsg, 16)] = wv
            for j in range(4):
                m = jnp.where(lo, wv[2 * j], wv[2 * j + 1])
                msgb[k, pl.ds(16 * j, 16)] = bs[k, pl.ds(16 * j, 16)] * m
        else:
            msgb[k, pl.ds(nmsg, 16)] = jnp.where(lane == 0, wv, 0.0)
            w0 = wv[0]
            for j in range(3):
                msgb[k, pl.ds(16 * j, 16)] = bs[k, pl.ds(16 * j, 16)] * w0

    bufs = ((bs0, bd0, ss0, sd0), (bs1, bd1, ss1, sd1),
            (bs2, bd2, ss2, sd2), (bs3, bd3, ss3, sd3))

    def chunk_quad(t, carry):
        ivs = []
        cps = []
        for u in range(4):
            cu = 4 * t + u
            iv_s = idx_s[pl.ds(16 * cu, 16)]
            iv_d = idx_d[pl.ds(16 * cu, 16)]
            bs, bd, ssem, dsem = bufs[u]
            cps.append((pltpu.async_copy(tab_hbm.at[iv_s], bs, ssem),
                        pltpu.async_copy(tab_hbm.at[iv_d], bd, dsem)))
            ivs.append(iv_d)
        for u in range(4):
            bs, bd, _, _ = bufs[u]
            cps[u][0].wait()
            cps[u][1].wait()
            for k in range(CHUNK):
                ebody(k, bs, bd)
            pltpu.sync_copy(msgb, acc_sh.at[ivs[u]], add=True)
        return carry

    lax.fori_loop(0, CHUNKS_PER_TILE // 4, chunk_quad, 0)
    plsc.subcore_barrier()

    for t in range(ROWS // CHUNK):
        pltpu.sync_copy(acc_sh.at[pl.ds(rbase + t * CHUNK, CHUNK)], msgb)
        pltpu.sync_copy(msgb, out_acc.at[c, pl.ds(rbase + t * CHUNK, CHUNK)])


def _make_sc_layer(tw, aw, eoff, nmsg, nheads):
    body = functools.partial(
        _sc_layer, aw=aw, eoff=eoff, nmsg=nmsg, nheads=nheads)
    return functools.partial(
        pl.kernel,
        out_type=[jax.ShapeDtypeStruct((2, NP, aw), jnp.float32)],
        mesh=_mesh,
        scratch_types=[
            pltpu.VMEM((PER_TILE,), jnp.int32),
            pltpu.VMEM((PER_TILE,), jnp.int32),
            pltpu.VMEM((CHUNK, tw), jnp.float32),
            pltpu.VMEM((CHUNK, tw), jnp.float32),
            pltpu.VMEM((CHUNK, tw), jnp.float32),
            pltpu.VMEM((CHUNK, tw), jnp.float32),
            pltpu.VMEM((CHUNK, tw), jnp.float32),
            pltpu.VMEM((CHUNK, tw), jnp.float32),
            pltpu.VMEM((CHUNK, tw), jnp.float32),
            pltpu.VMEM((CHUNK, tw), jnp.float32),
            pltpu.VMEM((CHUNK, aw), jnp.float32),
            pltpu.VMEM_SHARED((NP, aw), jnp.float32),
            pltpu.SemaphoreType.DMA,
            pltpu.SemaphoreType.DMA,
            pltpu.SemaphoreType.DMA,
            pltpu.SemaphoreType.DMA,
            pltpu.SemaphoreType.DMA,
            pltpu.SemaphoreType.DMA,
            pltpu.SemaphoreType.DMA,
            pltpu.SemaphoreType.DMA,
        ],
    )(body)


_layer1_edges = _make_sc_layer(tw=T1, aw=A1, eoff=64, nmsg=64, nheads=8)
_layer2_edges = _make_sc_layer(tw=T2, aw=A2, eoff=48, nmsg=48, nheads=1)


@jax.jit
def kernel(x, edge_index, W1, a_src1, a_dst1, b1, W2, a_src2, a_dst2, b2):
    f32 = jnp.float32
    x_p = jnp.pad(x, ((0, NP - N), (0, 0)))
    loops = jnp.arange(N, dtype=jnp.int32)
    npad = EPAD - (E + N)
    pad_idx = N + (jnp.arange(npad, dtype=jnp.int32) % NPAD_ROWS)
    src = jnp.concatenate([edge_index[0].astype(jnp.int32), loops, pad_idx])
    dst = jnp.concatenate([edge_index[1].astype(jnp.int32), loops, pad_idx])

    eye8 = jnp.eye(8, dtype=f32)
    As1 = jnp.einsum("hc,hg->hcg", a_src1, eye8).reshape(64, 8)
    Ad1 = jnp.einsum("hc,hg->hcg", a_dst1, eye8).reshape(64, 8)[:, ::-1]
    E8 = jnp.repeat(eye8, 8, axis=1)

    z1 = jnp.zeros((CHUNK, A1), f32)
    z2 = jnp.zeros((CHUNK, A2), f32)

    tab1 = pl.pallas_call(
        _tc_prep1,
        out_shape=jax.ShapeDtypeStruct((NP, T1), f32),
    )(x_p, W1, As1, Ad1)

    acc1, = _layer1_edges(src, dst, tab1, z1)

    tab2 = pl.pallas_call(
        _tc_mid,
        out_shape=jax.ShapeDtypeStruct((NP, T2), f32),
    )(acc1, b1, W2, a_src2.reshape(NC, 1), a_dst2.reshape(NC, 1), E8)

    acc2, = _layer2_edges(src, dst, tab2, z2)

    out = pl.pallas_call(
        _tc_final,
        out_shape=jax.ShapeDtypeStruct((N, NC), f32),
    )(acc2, b2)
    return out
